# baseline (speedup 1.0000x reference)
import jax
import jax.numpy as jnp
from jax import lax
from jax.experimental import pallas as pl
from jax.experimental.pallas import tpu as pltpu
from jax.experimental.pallas import tpu_sc as plsc

DIMS = 64
LANE = 128


def _build(batch: int, hist: int, nc: int, ns: int):
    nw = nc * ns
    bw = batch // nw
    k = bw // LANE
    nblk = hist
    nhalf = nblk // 2

    mesh = plsc.VectorSubcoreMesh(core_axis_name="c", subcore_axis_name="s")

    @jax.jit
    def run(fnums, table):
        fnums_t = jnp.transpose(fnums)
        table_t = lax.optimization_barrier(jnp.transpose(table))
        table_rm = jnp.transpose(table_t)

        @pl.kernel(
            out_type=jax.ShapeDtypeStruct((hist, batch, 2 * DIMS), jnp.float32),
            mesh=mesh,
            scratch_types=[
                pltpu.VMEM((2, bw), jnp.int32),
                pltpu.VMEM((2, bw, DIMS), jnp.float32),
                pltpu.SemaphoreType.DMA((2,)),
                pltpu.SemaphoreType.DMA((2,)),
            ],
            compiler_params=pltpu.CompilerParams(use_tc_tiling_on_sc=False),
        )
        def kern(fnums_hbm, table_hbm, out_hbm, idx_v, rows_v, gsem, osem):
            wid = lax.axis_index("s") * nc + lax.axis_index("c")
            b0 = wid * bw

            def gather_descs(s, make):
                for j in range(k):
                    make(
                        table_hbm.at[idx_v.at[s].at[pl.ds(j * LANE, LANE)]],
                        rows_v.at[s].at[pl.ds(j * LANE, LANE)],
                        gsem.at[s],
                    )

            def stage_and_fire(g, s):
                pltpu.sync_copy(fnums_hbm.at[g, pl.ds(b0, bw)], idx_v.at[s])
                gather_descs(s, pltpu.async_copy)

            def wait_gathers(s):
                gather_descs(
                    s, lambda a, b, c: pltpu.make_async_copy(a, b, c).wait())

            def start_out(g, s):
                pltpu.async_copy(
                    rows_v.at[s],
                    out_hbm.at[g, pl.ds(b0, bw), pl.ds(0, DIMS)],
                    osem.at[s],
                )

            def wait_out(g, s):
                pltpu.make_async_copy(
                    rows_v.at[s],
                    out_hbm.at[g, pl.ds(b0, bw), pl.ds(0, DIMS)],
                    osem.at[s],
                ).wait()

            stage_and_fire(0, 0)

            def body(i, carry):
                g0 = 2 * i
                g1 = g0 + 1

                @pl.when(i > 0)
                def _():
                    wait_out(g1 - 2, 1)
                stage_and_fire(g1, 1)
                wait_gathers(0)
                start_out(g0, 0)

                @pl.when(i < nhalf - 1)
                def _():
                    wait_out(g0, 0)
                    stage_and_fire(g0 + 2, 0)
                wait_gathers(1)
                start_out(g1, 1)
                return carry

            lax.fori_loop(0, nhalf, body, 0)
            wait_out(nblk - 2, 0)
            wait_out(nblk - 1, 1)

        padded = kern(fnums_t, table_rm)
        return jnp.transpose(padded[:, :, :DIMS], (1, 0, 2))

    return run


def kernel(fnums, table):
    batch, hist = fnums.shape
    info = plsc.get_sparse_core_info()
    run = _build(batch, hist, info.num_cores, info.num_subcores)
    return run(fnums, table)


# speedup vs baseline: 1.5643x; 1.5643x over previous
import jax
import jax.numpy as jnp
from jax import lax
from jax.experimental import pallas as pl
from jax.experimental.pallas import tpu as pltpu
from jax.experimental.pallas import tpu_sc as plsc

DIMS = 64
LANE = 128


def _build(batch: int, hist: int, nc: int, ns: int):
    nw = nc * ns
    bw = batch // nw
    k = bw // LANE
    nblk = hist
    nhalf = nblk // 2

    mesh = plsc.VectorSubcoreMesh(core_axis_name="c", subcore_axis_name="s")

    @jax.jit
    def run(fnums, table):
        fnums_t = jnp.transpose(fnums)
        table_t = lax.optimization_barrier(jnp.transpose(table))
        table_rm = jnp.transpose(table_t)

        @pl.kernel(
            out_type=jax.ShapeDtypeStruct((hist, batch, 2 * DIMS), jnp.float32),
            mesh=mesh,
            scratch_types=[
                pltpu.VMEM((2, bw), jnp.int32),
                pltpu.VMEM((2, bw, DIMS), jnp.float32),
                pltpu.SemaphoreType.DMA((2,)),
                pltpu.SemaphoreType.DMA((2,)),
            ],
            compiler_params=pltpu.CompilerParams(use_tc_tiling_on_sc=False),
        )
        def kern(fnums_hbm, table_hbm, out_hbm, idx_v, rows_v, gsem, osem):
            wid = lax.axis_index("s") * nc + lax.axis_index("c")
            b0 = wid * bw

            def gather_descs(s, make):
                for j in range(k):
                    make(
                        table_hbm.at[idx_v.at[s].at[pl.ds(j * LANE, LANE)]],
                        rows_v.at[s].at[pl.ds(j * LANE, LANE)],
                        gsem.at[s],
                    )

            def stage_and_fire(g, s):
                pltpu.sync_copy(fnums_hbm.at[g, pl.ds(b0, bw)], idx_v.at[s])
                gather_descs(s, pltpu.async_copy)

            def wait_gathers(s):
                gather_descs(
                    s, lambda a, b, c: pltpu.make_async_copy(a, b, c).wait())

            def start_out(g, s):
                pltpu.async_copy(
                    rows_v.at[s],
                    out_hbm.at[g, pl.ds(b0, bw), pl.ds(0, DIMS)],
                    osem.at[s],
                )

            def wait_out(g, s):
                pltpu.make_async_copy(
                    rows_v.at[s],
                    out_hbm.at[g, pl.ds(b0, bw), pl.ds(0, DIMS)],
                    osem.at[s],
                ).wait()

            stage_and_fire(0, 0)

            def body(i, carry):
                g0 = 2 * i
                g1 = g0 + 1

                @pl.when(i > 0)
                def _():
                    wait_out(g1 - 2, 1)
                stage_and_fire(g1, 1)
                wait_gathers(0)
                start_out(g0, 0)

                @pl.when(i < nhalf - 1)
                def _():
                    wait_out(g0, 0)
                    stage_and_fire(g0 + 2, 0)
                wait_gathers(1)
                start_out(g1, 1)
                return carry

            lax.fori_loop(0, nhalf, body, 0)
            wait_out(nblk - 2, 0)
            wait_out(nblk - 1, 1)

        padded = kern(fnums_t, table_rm)
        return jnp.transpose(padded, (1, 0, 2))[:, :, :DIMS]

    return run


def kernel(fnums, table):
    batch, hist = fnums.shape
    info = plsc.get_sparse_core_info()
    run = _build(batch, hist, info.num_cores, info.num_subcores)
    return run(fnums, table)
